# baseline (speedup 1.0000x reference)
import functools

import jax
import jax.numpy as jnp
from jax import lax
from jax.experimental import pallas as pl
from jax.experimental.pallas import tpu as pltpu
from jax.experimental.pallas import tpu_sc as plsc

NC = 2
NS = 16
L = 16
NW = NC * NS


def kernel(logits, topics, loga, b):
    B, C = logits.shape
    RPW = B // NW
    IG = 128
    G = 128
    NG = RPW // G
    NB = 2

    mesh = plsc.VectorSubcoreMesh(core_axis_name="c", subcore_axis_name="s")

    @functools.partial(
        pl.kernel,
        mesh=mesh,
        out_type=jax.ShapeDtypeStruct((B, C), jnp.float32),
        scratch_types=[
            pltpu.VMEM((RPW,), jnp.int32),
            pltpu.VMEM((RPW,), jnp.float32),
            pltpu.VMEM((RPW,), jnp.float32),
            pltpu.VMEM((NB, G, C), jnp.float32),
            pltpu.VMEM((NB, G, C), jnp.float32),
            pltpu.VMEM((NB, G, C), jnp.float32),
            pltpu.SemaphoreType.DMA,
            pltpu.SemaphoreType.DMA((NB,)),
            pltpu.SemaphoreType.DMA((NB,)),
            pltpu.SemaphoreType.DMA((NB,)),
        ],
    )
    def run(logits_hbm, topics_hbm, loga_hbm, b_hbm, out_hbm,
            idx_v, sraw_v, sc_v, brow_v, lrow_v, orow_v,
            sem_s, sem_b, sem_l, sem_o):
        wid = lax.axis_index("s") * NC + lax.axis_index("c")
        base = wid * RPW
        pltpu.sync_copy(topics_hbm.at[pl.ds(base, RPW)], idx_v)
        scale_copies = [
            pltpu.async_copy(loga_hbm.at[idx_v.at[pl.ds(k * IG, IG)]],
                             sraw_v.at[pl.ds(k * IG, IG)], sem_s)
            for k in range(RPW // IG)
        ]

        def in_copies(j, k):
            ro = pl.multiple_of(j * G, G)
            return (
                pltpu.make_async_copy(
                    b_hbm.at[idx_v.at[pl.ds(ro, G)]], brow_v.at[k],
                    sem_b.at[k]),
                pltpu.make_async_copy(
                    logits_hbm.at[pl.ds(base + ro, G)], lrow_v.at[k],
                    sem_l.at[k]),
            )

        H = G // 2

        def out_copy(j, k, h):
            ro = pl.multiple_of(j * G, G) + h * H
            return pltpu.make_async_copy(
                orow_v.at[k, pl.ds(h * H, H)],
                out_hbm.at[pl.ds(base + ro, H)], sem_o.at[k])

        for k in range(NB):
            for cp in in_copies(k, k):
                cp.start()
        for cp in scale_copies:
            cp.wait()

        def exp_body(i, carry):
            o = pl.multiple_of(i * L, L)
            sc_v[pl.ds(o, L)] = jnp.exp(sraw_v[pl.ds(o, L)])
            return carry

        lax.fori_loop(0, RPW // L, exp_body, 0, unroll=2)

        def ring_body(it, carry):
            for k in range(NB):
                j = it * NB + k
                for cp in in_copies(j, k):
                    cp.wait()

                @pl.when(it > 0)
                def _():
                    for h in range(2):
                        out_copy(j - NB, k, h).wait()

                def blk_body(bk, carry2):
                    sv = sc_v[pl.ds(j * G + bk * L, L)]
                    for rr in range(L):
                        s = sv[rr]
                        for c in range(C // L):
                            orow_v[k, bk * L + rr, pl.ds(c * L, L)] = (
                                lrow_v[k, bk * L + rr, pl.ds(c * L, L)] * s
                                + brow_v[k, bk * L + rr, pl.ds(c * L, L)])
                    return carry2

                for h in range(2):
                    lax.fori_loop(h * (G // L // 2), (h + 1) * (G // L // 2),
                                  blk_body, 0, unroll=2)
                    out_copy(j, k, h).start()

                @pl.when(it < NG // NB - 1)
                def _():
                    for cp in in_copies(j + NB, k):
                        cp.start()
            return carry

        lax.fori_loop(0, NG // NB, ring_body, 0)
        for k in range(NB):
            for h in range(2):
                out_copy(NG - NB + k, k, h).wait()

    return run(logits, topics, loga, b)


# speedup vs baseline: 1.1960x; 1.1960x over previous
import functools

import jax
import jax.numpy as jnp
from jax import lax
from jax.experimental import pallas as pl
from jax.experimental.pallas import tpu as pltpu
from jax.experimental.pallas import tpu_sc as plsc

NC = 2
NS = 16
L = 16
NW = NC * NS


def kernel(logits, topics, loga, b):
    B, C = logits.shape
    RPW = B // NW
    IG = 128
    G = 128
    NG = RPW // G
    NB = 2

    mesh = plsc.VectorSubcoreMesh(core_axis_name="c", subcore_axis_name="s")

    @functools.partial(
        pl.kernel,
        mesh=mesh,
        out_type=jax.ShapeDtypeStruct((B, C), jnp.float32),
        scratch_types=[
            pltpu.VMEM((RPW,), jnp.int32),
            pltpu.VMEM((RPW,), jnp.float32),
            pltpu.VMEM((RPW,), jnp.float32),
            pltpu.VMEM((NB, G, C), jnp.float32),
            pltpu.VMEM((NB, G, C), jnp.float32),
            pltpu.VMEM((NB, G, C), jnp.float32),
            pltpu.SemaphoreType.DMA,
            pltpu.SemaphoreType.DMA((NB,)),
            pltpu.SemaphoreType.DMA((NB,)),
            pltpu.SemaphoreType.DMA((NB,)),
        ],
    )
    def run(logits_hbm, topics_hbm, loga_hbm, b_hbm, out_hbm,
            idx_v, sraw_v, sc_v, brow_v, lrow_v, orow_v,
            sem_s, sem_b, sem_l, sem_o):
        wid = lax.axis_index("s") * NC + lax.axis_index("c")
        base = wid * RPW
        pltpu.sync_copy(topics_hbm.at[pl.ds(base, RPW)], idx_v)
        scale_copies = [
            pltpu.async_copy(loga_hbm.at[idx_v.at[pl.ds(k * IG, IG)]],
                             sraw_v.at[pl.ds(k * IG, IG)], sem_s)
            for k in range(RPW // IG)
        ]

        def in_copies(j, k):
            ro = pl.multiple_of(j * G, G)
            return (
                pltpu.make_async_copy(
                    b_hbm.at[idx_v.at[pl.ds(ro, G)]], brow_v.at[k],
                    sem_b.at[k]),
                pltpu.make_async_copy(
                    logits_hbm.at[pl.ds(base + ro, G)], lrow_v.at[k],
                    sem_l.at[k]),
            )

        def out_copy(j, k):
            ro = pl.multiple_of(j * G, G)
            return pltpu.make_async_copy(
                orow_v.at[k], out_hbm.at[pl.ds(base + ro, G)], sem_o.at[k])

        for k in range(NB):
            for cp in in_copies(k, k):
                cp.start()
        for cp in scale_copies:
            cp.wait()
        for i in range(RPW // L):
            sc_v[pl.ds(i * L, L)] = jnp.exp(sraw_v[pl.ds(i * L, L)])

        def ring_body(it, carry):
            for k in range(NB):
                j = it * NB + k
                for cp in in_copies(j, k):
                    cp.wait()

                @pl.when(it > 0)
                def _():
                    out_copy(j - NB, k).wait()

                def blk_body(bk, carry2):
                    sv = sc_v[pl.ds(j * G + bk * L, L)]
                    for rr in range(L):
                        s = sv[rr]
                        for c in range(C // L):
                            orow_v[k, bk * L + rr, pl.ds(c * L, L)] = (
                                lrow_v[k, bk * L + rr, pl.ds(c * L, L)] * s
                                + brow_v[k, bk * L + rr, pl.ds(c * L, L)])
                    return carry2

                lax.fori_loop(0, G // L, blk_body, 0, unroll=2)
                out_copy(j, k).start()

                @pl.when(it < NG // NB - 1)
                def _():
                    for cp in in_copies(j + NB, k):
                        cp.start()
            return carry

        lax.fori_loop(0, NG // NB, ring_body, 0)
        for k in range(NB):
            out_copy(NG - NB + k, k).wait()

    return run(logits, topics, loga, b)
